# baseline (speedup 1.0000x reference)
import functools

import jax
import jax.numpy as jnp
from jax import lax
from jax.experimental import pallas as pl
from jax.experimental.pallas import tpu as pltpu
from jax.experimental.pallas import tpu_sc as plsc

N = 10000
D = 128
E = 320000
NC = 2
NS = 16
NW = NC * NS
CH = 97
CPW = 104
ROWS2D = NW * CPW
EPAD = ROWS2D * CH
NDEG = 10240
DEG_SL = NDEG // NS
NACC = 10008
ACC_SL = 632
ACC_SL_LAST = NACC - (NS - 1) * ACC_SL

def _mesh():
    return plsc.VectorSubcoreMesh(
        core_axis_name="c", subcore_axis_name="s",
        num_cores=NC, num_subcores=NS)


def _sc_degree_body(dst_hbm, zero_hbm, out_hbm, idx_v, ones_v, acc):
    c = lax.axis_index("c")
    s = lax.axis_index("s")
    w = c * NS + s
    pltpu.sync_copy(zero_hbm.at[pl.ds(s * DEG_SL, DEG_SL)],
                    acc.at[pl.ds(s * DEG_SL, DEG_SL)])
    pltpu.sync_copy(dst_hbm.at[pl.ds(w * CPW, CPW)], idx_v)
    for k in range(8):
        ones_v[pl.ds(k * 16, 16)] = jnp.ones((16,), jnp.float32)
    plsc.subcore_barrier()

    def body(j, carry):
        pltpu.sync_copy(ones_v.at[pl.ds(0, CH)], acc.at[idx_v.at[j]], add=True)
        return carry

    lax.fori_loop(0, CPW, body, 0)
    plsc.subcore_barrier()
    pltpu.sync_copy(acc.at[pl.ds(s * DEG_SL, DEG_SL)],
                    out_hbm.at[c, pl.ds(s * DEG_SL, DEG_SL)])


NB = 3
NI = 6


def _sc_aggregate_body(tab_hbm, src_hbm, dst_hbm, zero_hbm, out_hbm,
                       sidx, didx, rows_v, acc,
                       is0, is1, is2, gs0, gs1, ss0, ss1):
    isem = (is0, is1, is2)
    gsem = (gs0, gs1)
    ssem = (ss0, ss1)
    c = lax.axis_index("c")
    s = lax.axis_index("s")
    base = (c * NS + s) * CPW

    def start_i(sl, j):
        pltpu.async_copy(src_hbm.at[base + j], sidx.at[sl], isem[sl % 3])
        pltpu.async_copy(dst_hbm.at[base + j], didx.at[sl], isem[sl % 3])

    def wait_i(sl, j):
        pltpu.make_async_copy(src_hbm.at[base + j], sidx.at[sl],
                              isem[sl % 3]).wait()
        pltpu.make_async_copy(dst_hbm.at[base + j], didx.at[sl],
                              isem[sl % 3]).wait()

    def start_g(b, sl, p):
        pltpu.async_copy(tab_hbm.at[sidx.at[sl]], rows_v.at[b], gsem[p])

    def wait_g(b, sl, p):
        pltpu.make_async_copy(tab_hbm.at[sidx.at[sl]], rows_v.at[b],
                              gsem[p]).wait()

    def start_s(b, sl, p):
        pltpu.async_copy(rows_v.at[b], acc.at[didx.at[sl]], ssem[p],
                         add=True)

    def wait_s(b, sl, p):
        pltpu.make_async_copy(rows_v.at[b], acc.at[didx.at[sl]],
                              ssem[p]).wait()

    for k in range(3):
        start_i(k, k)

    @pl.when(s < NS - 1)
    def _():
        pltpu.sync_copy(zero_hbm.at[pl.ds(s * ACC_SL, ACC_SL)],
                        acc.at[pl.ds(s * ACC_SL, ACC_SL)])

    @pl.when(s == NS - 1)
    def _():
        pltpu.sync_copy(zero_hbm.at[pl.ds((NS - 1) * ACC_SL, ACC_SL_LAST)],
                        acc.at[pl.ds((NS - 1) * ACC_SL, ACC_SL_LAST)])

    wait_i(0, 0)
    start_g(0, 0, 0)
    start_i(3, 3)
    wait_i(1, 1)
    start_g(1, 1, 1)
    start_i(4, 4)
    plsc.subcore_barrier()

    wait_g(0, 0, 0)
    start_s(0, 0, 0)
    wait_i(2, 2)
    start_g(2, 2, 0)
    start_i(5, 5)

    def grp_body(grp, carry):
        for k in range(NI):
            j = grp * NI + k + 1
            sl = (k + 1) % NI
            b = (k + 1) % NB
            p = (k + 1) % 2
            slp = k % NI
            bp = k % NB
            wait_g(b, sl, p)
            wait_s(bp, slp, k % 2)
            start_s(b, sl, p)

            def next_g(j=j, k=k, p=p):
                wait_i((k + 3) % NI, j + 2)
                start_g((k + 3) % NB, (k + 3) % NI, p)

            pl.when(j + 2 < CPW)(next_g)
            pl.when(j + 5 < CPW)(lambda slp=slp, j=j: start_i(slp, j + 5))
        return carry

    lax.fori_loop(0, (CPW - 2) // NI, grp_body, 0)
    wait_g((CPW - 1) % NB, (CPW - 1) % NI, (CPW - 1) % 2)
    wait_s((CPW - 2) % NB, (CPW - 2) % NI, (CPW - 2) % 2)
    start_s((CPW - 1) % NB, (CPW - 1) % NI, (CPW - 1) % 2)
    wait_s((CPW - 1) % NB, (CPW - 1) % NI, (CPW - 1) % 2)
    plsc.subcore_barrier()

    @pl.when(s < NS - 1)
    def _():
        pltpu.sync_copy(acc.at[pl.ds(s * ACC_SL, ACC_SL)],
                        out_hbm.at[c, pl.ds(s * ACC_SL, ACC_SL)])

    @pl.when(s == NS - 1)
    def _():
        pltpu.sync_copy(acc.at[pl.ds((NS - 1) * ACC_SL, ACC_SL_LAST)],
                        out_hbm.at[c, pl.ds((NS - 1) * ACC_SL, ACC_SL_LAST)])


@functools.cache
def _sc_degree():
    return pl.kernel(
        _sc_degree_body,
        out_type=jax.ShapeDtypeStruct((NC, NDEG), jnp.float32),
        mesh=_mesh(),
        scratch_types=[
            pltpu.VMEM((CPW, CH), jnp.int32),
            pltpu.VMEM((128,), jnp.float32),
            pltpu.VMEM_SHARED((NDEG,), jnp.float32),
        ],
    )


@functools.cache
def _sc_aggregate():
    return pl.kernel(
        _sc_aggregate_body,
        out_type=jax.ShapeDtypeStruct((NC, NACC, D), jnp.float32),
        mesh=_mesh(),
        scratch_types=[
            pltpu.VMEM((NI, CH), jnp.int32),
            pltpu.VMEM((NI, CH), jnp.int32),
            pltpu.VMEM((NB, CH, D), jnp.float32),
            pltpu.VMEM_SHARED((NACC, D), jnp.float32),
        ] + [pltpu.SemaphoreType.DMA] * 7,
    )


_BLK = 1000
_GRID = N // _BLK


def _dis(dp_ref):
    return lax.rsqrt(dp_ref[:, 0:1] + dp_ref[:, 1:2] + 1.0)


def _tc1_body(x_ref, w_ref, dp_ref, o_ref):
    o_ref[...] = jnp.dot(x_ref[...], w_ref[...],
                         preferred_element_type=jnp.float32) * _dis(dp_ref)


def _tc2_body(p_ref, g1_ref, dp_ref, b_ref, w_ref, o_ref):
    dis = _dis(dp_ref)
    h = jnp.maximum((p_ref[0] + p_ref[1] + g1_ref[...]) * dis + b_ref[...], 0.0)
    o_ref[...] = jnp.dot(h, w_ref[...],
                         preferred_element_type=jnp.float32) * dis


def _tc3_body(p_ref, g2_ref, dp_ref, b_ref, o_ref):
    o_ref[...] = (p_ref[0] + p_ref[1] + g2_ref[...]) * _dis(dp_ref) + b_ref[...]


_row_spec = pl.BlockSpec((_BLK, D), lambda i: (i, 0))
_dp_spec = pl.BlockSpec((_BLK, 2), lambda i: (i, 0))
_w_spec = pl.BlockSpec((D, D), lambda i: (0, 0))
_b_spec = pl.BlockSpec((1, D), lambda i: (0, 0))
_p_spec = pl.BlockSpec((NC, _BLK, D), lambda i: (0, i, 0))
_out_sds = jax.ShapeDtypeStruct((N, D), jnp.float32)


def _tc1(x, W1, dp):
    return pl.pallas_call(
        _tc1_body, grid=(_GRID,),
        in_specs=[_row_spec, _w_spec, _dp_spec],
        out_specs=_row_spec, out_shape=_out_sds)(x, W1, dp)


def _tc2(p, g1, dp, b1, W2):
    return pl.pallas_call(
        _tc2_body, grid=(_GRID,),
        in_specs=[_p_spec, _row_spec, _dp_spec, _b_spec, _w_spec],
        out_specs=_row_spec, out_shape=_out_sds)(p, g1, dp, b1, W2)


def _tc3(p, g2, dp, b2):
    return pl.pallas_call(
        _tc3_body, grid=(_GRID,),
        in_specs=[_p_spec, _row_spec, _dp_spec, _b_spec],
        out_specs=_row_spec, out_shape=_out_sds)(p, g2, dp, b2)


def kernel(x, edge_index, W1, b1, W2, b2):
    ei = edge_index.astype(jnp.int32)
    pad = EPAD - E
    src2d = jnp.concatenate(
        [ei[0], jnp.zeros((pad,), jnp.int32)]).reshape(ROWS2D, CH)
    dst2d = jnp.concatenate(
        [ei[1], jnp.full((pad,), N, jnp.int32)]).reshape(ROWS2D, CH)
    zrows = jnp.zeros((NACC, D), jnp.float32)
    zdeg = jnp.zeros((NDEG,), jnp.float32)
    b1r = b1.reshape(1, D)
    b2r = b2.reshape(1, D)

    degp = _sc_degree()(dst2d, zdeg)
    dp = degp.T

    g1 = _tc1(x, W1, dp)
    p1 = _sc_aggregate()(g1, src2d, dst2d, zrows)
    g2 = _tc2(p1, g1, dp, b1r, W2)
    p2 = _sc_aggregate()(g2, src2d, dst2d, zrows)
    return _tc3(p2, g2, dp, b2r)


# speedup vs baseline: 1.0309x; 1.0309x over previous
import functools

import jax
import jax.numpy as jnp
from jax import lax
from jax.experimental import pallas as pl
from jax.experimental.pallas import tpu as pltpu
from jax.experimental.pallas import tpu_sc as plsc

N = 10000
D = 128
E = 320000
NC = 2
NS = 16
NW = NC * NS
CH = 97
CPW = 104
ROWS2D = NW * CPW
EPAD = ROWS2D * CH
NDEG = 10240
DEG_SL = NDEG // NS
NACC = 10008
ACC_SL = 632
ACC_SL_LAST = NACC - (NS - 1) * ACC_SL

def _mesh():
    return plsc.VectorSubcoreMesh(
        core_axis_name="c", subcore_axis_name="s",
        num_cores=NC, num_subcores=NS)


def _sc_degree_body(dst_hbm, zero_hbm, out_hbm, idx_v, ones_v, acc):
    c = lax.axis_index("c")
    s = lax.axis_index("s")
    w = c * NS + s
    pltpu.sync_copy(zero_hbm.at[pl.ds(s * DEG_SL, DEG_SL)],
                    acc.at[pl.ds(s * DEG_SL, DEG_SL)])
    pltpu.sync_copy(dst_hbm.at[pl.ds(w * CPW, CPW)], idx_v)
    for k in range(8):
        ones_v[pl.ds(k * 16, 16)] = jnp.ones((16,), jnp.float32)
    plsc.subcore_barrier()

    def body(j, carry):
        pltpu.sync_copy(ones_v.at[pl.ds(0, CH)], acc.at[idx_v.at[j]], add=True)
        return carry

    lax.fori_loop(0, CPW, body, 0)
    plsc.subcore_barrier()
    pltpu.sync_copy(acc.at[pl.ds(s * DEG_SL, DEG_SL)],
                    out_hbm.at[c, pl.ds(s * DEG_SL, DEG_SL)])


NB = 3
NI = 6


def _sc_aggregate_body(tab_hbm, src_hbm, dst_hbm, zero_hbm, out_hbm,
                       sidx, didx, rows_v, acc,
                       is0, is1, is2, gs0, gs1, ss0, ss1):
    isem = (is0, is1, is2)
    gsem = (gs0, gs1)
    ssem = (ss0, ss1)
    c = lax.axis_index("c")
    s = lax.axis_index("s")
    base = (c * NS + s) * CPW

    def start_i(sl, j):
        pltpu.async_copy(src_hbm.at[base + j], sidx.at[sl], isem[sl % 3])
        pltpu.async_copy(dst_hbm.at[base + j], didx.at[sl], isem[sl % 3])

    def wait_i(sl, j):
        pltpu.make_async_copy(src_hbm.at[base + j], sidx.at[sl],
                              isem[sl % 3]).wait()
        pltpu.make_async_copy(dst_hbm.at[base + j], didx.at[sl],
                              isem[sl % 3]).wait()

    def start_g(b, sl, p):
        pltpu.async_copy(tab_hbm.at[sidx.at[sl]], rows_v.at[b], gsem[p])

    def wait_g(b, sl, p):
        pltpu.make_async_copy(tab_hbm.at[sidx.at[sl]], rows_v.at[b],
                              gsem[p]).wait()

    def start_s(b, sl, p):
        pltpu.async_copy(rows_v.at[b], acc.at[didx.at[sl]], ssem[p],
                         add=True)

    def wait_s(b, sl, p):
        pltpu.make_async_copy(rows_v.at[b], acc.at[didx.at[sl]],
                              ssem[p]).wait()

    for k in range(3):
        start_i(k, k)

    @pl.when(s < NS - 1)
    def _():
        pltpu.sync_copy(zero_hbm.at[pl.ds(s * ACC_SL, ACC_SL)],
                        acc.at[pl.ds(s * ACC_SL, ACC_SL)])

    @pl.when(s == NS - 1)
    def _():
        pltpu.sync_copy(zero_hbm.at[pl.ds((NS - 1) * ACC_SL, ACC_SL_LAST)],
                        acc.at[pl.ds((NS - 1) * ACC_SL, ACC_SL_LAST)])

    wait_i(0, 0)
    start_g(0, 0, 0)
    start_i(3, 3)
    wait_i(1, 1)
    start_g(1, 1, 1)
    start_i(4, 4)
    plsc.subcore_barrier()

    wait_g(0, 0, 0)
    start_s(0, 0, 0)
    wait_i(2, 2)
    start_g(2, 2, 0)
    start_i(5, 5)

    def grp_body(grp, carry):
        for k in range(NI):
            j = grp * NI + k + 1
            sl = (k + 1) % NI
            b = (k + 1) % NB
            p = (k + 1) % 2
            slp = k % NI
            bp = k % NB
            wait_g(b, sl, p)
            wait_s(bp, slp, k % 2)
            start_s(b, sl, p)

            def next_g(j=j, k=k, p=p):
                wait_i((k + 3) % NI, j + 2)
                start_g((k + 3) % NB, (k + 3) % NI, p)

            pl.when(j + 2 < CPW)(next_g)
            pl.when(j + 5 < CPW)(lambda slp=slp, j=j: start_i(slp, j + 5))
        return carry

    lax.fori_loop(0, (CPW - 2) // NI, grp_body, 0)
    wait_g((CPW - 1) % NB, (CPW - 1) % NI, (CPW - 1) % 2)
    wait_s((CPW - 2) % NB, (CPW - 2) % NI, (CPW - 2) % 2)
    start_s((CPW - 1) % NB, (CPW - 1) % NI, (CPW - 1) % 2)
    wait_s((CPW - 1) % NB, (CPW - 1) % NI, (CPW - 1) % 2)
    plsc.subcore_barrier()

    @pl.when(s < NS - 1)
    def _():
        pltpu.sync_copy(acc.at[pl.ds(s * ACC_SL, ACC_SL)],
                        out_hbm.at[c, pl.ds(s * ACC_SL, ACC_SL)])

    @pl.when(s == NS - 1)
    def _():
        pltpu.sync_copy(acc.at[pl.ds((NS - 1) * ACC_SL, ACC_SL_LAST)],
                        out_hbm.at[c, pl.ds((NS - 1) * ACC_SL, ACC_SL_LAST)])


@functools.cache
def _sc_degree():
    return pl.kernel(
        _sc_degree_body,
        out_type=jax.ShapeDtypeStruct((NC, NDEG), jnp.float32),
        mesh=_mesh(),
        scratch_types=[
            pltpu.VMEM((CPW, CH), jnp.int32),
            pltpu.VMEM((128,), jnp.float32),
            pltpu.VMEM_SHARED((NDEG,), jnp.float32),
        ],
    )


@functools.cache
def _sc_aggregate():
    return pl.kernel(
        _sc_aggregate_body,
        out_type=jax.ShapeDtypeStruct((NC, NACC, D), jnp.float32),
        mesh=_mesh(),
        scratch_types=[
            pltpu.VMEM((NI, CH), jnp.int32),
            pltpu.VMEM((NI, CH), jnp.int32),
            pltpu.VMEM((NB, CH, D), jnp.float32),
            pltpu.VMEM_SHARED((NACC, D), jnp.float32),
        ] + [pltpu.SemaphoreType.DMA] * 7,
    )


_BLK = 1000
_GRID = N // _BLK


def _dis(dp_ref):
    return lax.rsqrt(dp_ref[:, 0:1] + dp_ref[:, 1:2] + 1.0)


def _tc1_body(x_ref, w_ref, dp_ref, o_ref):
    o_ref[...] = jnp.dot(x_ref[...], w_ref[...],
                         preferred_element_type=jnp.float32) * _dis(dp_ref)


def _tc2_body(p_ref, g1_ref, dp_ref, b_ref, w_ref, o_ref):
    dis = _dis(dp_ref)
    h = jnp.maximum((p_ref[0] + p_ref[1] + g1_ref[...]) * dis + b_ref[...], 0.0)
    o_ref[...] = jnp.dot(h, w_ref[...],
                         preferred_element_type=jnp.float32) * dis


def _tc3_body(p_ref, g2_ref, dp_ref, b_ref, o_ref):
    o_ref[...] = (p_ref[0] + p_ref[1] + g2_ref[...]) * _dis(dp_ref) + b_ref[...]


_row_spec = pl.BlockSpec((_BLK, D), lambda i: (i, 0))
_dp_spec = pl.BlockSpec((_BLK, 2), lambda i: (i, 0))
_w_spec = pl.BlockSpec((D, D), lambda i: (0, 0))
_b_spec = pl.BlockSpec((1, D), lambda i: (0, 0))
_p_spec = pl.BlockSpec((NC, _BLK, D), lambda i: (0, i, 0))
_out_sds = jax.ShapeDtypeStruct((N, D), jnp.float32)


def _tc1(x, W1, dp):
    return pl.pallas_call(
        _tc1_body, grid=(_GRID,),
        in_specs=[_row_spec, _w_spec, _dp_spec],
        out_specs=_row_spec, out_shape=_out_sds)(x, W1, dp)


def _tc2(p, g1, dp, b1, W2):
    return pl.pallas_call(
        _tc2_body, grid=(_GRID,),
        in_specs=[_p_spec, _row_spec, _dp_spec, _b_spec, _w_spec],
        out_specs=_row_spec, out_shape=_out_sds)(p, g1, dp, b1, W2)


def _tc3(p, g2, dp, b2):
    return pl.pallas_call(
        _tc3_body, grid=(_GRID,),
        in_specs=[_p_spec, _row_spec, _dp_spec, _b_spec],
        out_specs=_row_spec, out_shape=_out_sds)(p, g2, dp, b2)


def kernel(x, edge_index, W1, b1, W2, b2):
    ei = edge_index.astype(jnp.int32)
    pad = EPAD - E
    src2d = jnp.concatenate(
        [ei[0], jnp.zeros((pad,), jnp.int32)]).reshape(ROWS2D, CH)
    dst2d = jnp.concatenate(
        [ei[1], N + (jnp.arange(pad, dtype=jnp.int32) % (NACC - N))]
    ).reshape(ROWS2D, CH)
    zrows = jnp.zeros((NACC, D), jnp.float32)
    zdeg = jnp.zeros((NDEG,), jnp.float32)
    b1r = b1.reshape(1, D)
    b2r = b2.reshape(1, D)

    degp = _sc_degree()(dst2d, zdeg)
    dp = degp.T

    g1 = _tc1(x, W1, dp)
    p1 = _sc_aggregate()(g1, src2d, dst2d, zrows)
    g2 = _tc2(p1, g1, dp, b1r, W2)
    p2 = _sc_aggregate()(g2, src2d, dst2d, zrows)
    return _tc3(p2, g2, dp, b2r)


# speedup vs baseline: 1.7554x; 1.7027x over previous
import functools

import jax
import jax.numpy as jnp
from jax import lax
from jax.experimental import pallas as pl
from jax.experimental.pallas import tpu as pltpu
from jax.experimental.pallas import tpu_sc as plsc

N = 10000
D = 128
E = 320000
NC = 2
NS = 16
NW = NC * NS
CH = 125
CPW = (E // NW) // CH
ROWS2D = E // CH
NDEG = 10240
DEG_SL = NDEG // NS
NACC = 10112
ACC_SL = NACC // NS

def _mesh():
    return plsc.VectorSubcoreMesh(
        core_axis_name="c", subcore_axis_name="s",
        num_cores=NC, num_subcores=NS)


def _sc_degree_body(dst_hbm, zero_hbm, out_hbm, idx_v, ones_v, acc):
    c = lax.axis_index("c")
    s = lax.axis_index("s")
    w = c * NS + s
    pltpu.sync_copy(zero_hbm.at[pl.ds(s * DEG_SL, DEG_SL)],
                    acc.at[pl.ds(s * DEG_SL, DEG_SL)])
    pltpu.sync_copy(dst_hbm.at[pl.ds(w * CPW, CPW)], idx_v)
    for k in range(8):
        ones_v[pl.ds(k * 16, 16)] = jnp.ones((16,), jnp.float32)
    plsc.subcore_barrier()

    def body(j, carry):
        pltpu.sync_copy(ones_v.at[pl.ds(0, CH)], acc.at[idx_v.at[j]], add=True)
        return carry

    lax.fori_loop(0, CPW, body, 0)
    plsc.subcore_barrier()
    pltpu.sync_copy(acc.at[pl.ds(s * DEG_SL, DEG_SL)],
                    out_hbm.at[c, pl.ds(s * DEG_SL, DEG_SL)])


NB = 2
NI = 4


def _sc_aggregate_body(tab_hbm, src_hbm, dst_hbm, zero_hbm, out_hbm,
                       sidx, didx, rows_v, acc,
                       is0, is1, is2, is3, gs0, gs1, ss0, ss1):
    isem = (is0, is1, is2, is3)
    gsem = (gs0, gs1)
    ssem = (ss0, ss1)
    c = lax.axis_index("c")
    s = lax.axis_index("s")
    base = (c * NS + s) * CPW

    def start_i(sl, j):
        pltpu.async_copy(src_hbm.at[base + j], sidx.at[sl], isem[sl])
        pltpu.async_copy(dst_hbm.at[base + j], didx.at[sl], isem[sl])

    def wait_i(sl, j):
        pltpu.make_async_copy(src_hbm.at[base + j], sidx.at[sl],
                              isem[sl]).wait()
        pltpu.make_async_copy(dst_hbm.at[base + j], didx.at[sl],
                              isem[sl]).wait()

    def start_g(b, sl):
        pltpu.async_copy(tab_hbm.at[sidx.at[sl]], rows_v.at[b], gsem[b])

    def wait_g(b, sl):
        pltpu.make_async_copy(tab_hbm.at[sidx.at[sl]], rows_v.at[b],
                              gsem[b]).wait()

    def start_s(b, sl):
        pltpu.async_copy(rows_v.at[b], acc.at[didx.at[sl]], ssem[b],
                         add=True)

    def wait_s(b, sl):
        pltpu.make_async_copy(rows_v.at[b], acc.at[didx.at[sl]],
                              ssem[b]).wait()

    for k in range(NI):
        start_i(k, k)
    pltpu.sync_copy(zero_hbm.at[pl.ds(s * ACC_SL, ACC_SL)],
                    acc.at[pl.ds(s * ACC_SL, ACC_SL)])
    for b in range(NB):
        wait_i(b, b)
        start_g(b, b)
    plsc.subcore_barrier()

    def grp_body(grp, carry):
        for k in range(NI):
            j = grp * NI + k
            b = k % NB
            wait_g(b, k)
            start_s(b, k)
            wait_s(b, k)
            pl.when(j + NI < CPW)(lambda k=k, j=j: start_i(k, j + NI))

            def next_g(j=j, b=b, k=k):
                wait_i((k + NB) % NI, j + NB)
                start_g(b, (k + NB) % NI)

            pl.when(j + NB < CPW)(next_g)
        return carry

    lax.fori_loop(0, CPW // NI, grp_body, 0)
    plsc.subcore_barrier()
    pltpu.sync_copy(acc.at[pl.ds(s * ACC_SL, ACC_SL)],
                    out_hbm.at[c, pl.ds(s * ACC_SL, ACC_SL)])


@functools.cache
def _sc_degree():
    return pl.kernel(
        _sc_degree_body,
        out_type=jax.ShapeDtypeStruct((NC, NDEG), jnp.float32),
        mesh=_mesh(),
        scratch_types=[
            pltpu.VMEM((CPW, CH), jnp.int32),
            pltpu.VMEM((128,), jnp.float32),
            pltpu.VMEM_SHARED((NDEG,), jnp.float32),
        ],
    )


@functools.cache
def _sc_aggregate():
    return pl.kernel(
        _sc_aggregate_body,
        out_type=jax.ShapeDtypeStruct((NC, NACC, D), jnp.float32),
        mesh=_mesh(),
        scratch_types=[
            pltpu.VMEM((NI, CH), jnp.int32),
            pltpu.VMEM((NI, CH), jnp.int32),
            pltpu.VMEM((NB, CH, D), jnp.float32),
            pltpu.VMEM_SHARED((NACC, D), jnp.float32),
        ] + [pltpu.SemaphoreType.DMA] * (NI + 2 * NB),
    )


_BLK = 1000
_GRID = N // _BLK


def _dis(dp_ref):
    return lax.rsqrt(dp_ref[:, 0:1] + dp_ref[:, 1:2] + 1.0)


def _tc1_body(x_ref, w_ref, dp_ref, o_ref):
    o_ref[...] = jnp.dot(x_ref[...], w_ref[...],
                         preferred_element_type=jnp.float32) * _dis(dp_ref)


def _tc2_body(p_ref, g1_ref, dp_ref, b_ref, w_ref, o_ref):
    dis = _dis(dp_ref)
    h = jnp.maximum((p_ref[0] + p_ref[1] + g1_ref[...]) * dis + b_ref[...], 0.0)
    o_ref[...] = jnp.dot(h, w_ref[...],
                         preferred_element_type=jnp.float32) * dis


def _tc3_body(p_ref, g2_ref, dp_ref, b_ref, o_ref):
    o_ref[...] = (p_ref[0] + p_ref[1] + g2_ref[...]) * _dis(dp_ref) + b_ref[...]


_row_spec = pl.BlockSpec((_BLK, D), lambda i: (i, 0))
_dp_spec = pl.BlockSpec((_BLK, 2), lambda i: (i, 0))
_w_spec = pl.BlockSpec((D, D), lambda i: (0, 0))
_b_spec = pl.BlockSpec((1, D), lambda i: (0, 0))
_p_spec = pl.BlockSpec((NC, _BLK, D), lambda i: (0, i, 0))
_out_sds = jax.ShapeDtypeStruct((N, D), jnp.float32)


def _tc1(x, W1, dp):
    return pl.pallas_call(
        _tc1_body, grid=(_GRID,),
        in_specs=[_row_spec, _w_spec, _dp_spec],
        out_specs=_row_spec, out_shape=_out_sds)(x, W1, dp)


def _tc2(p, g1, dp, b1, W2):
    return pl.pallas_call(
        _tc2_body, grid=(_GRID,),
        in_specs=[_p_spec, _row_spec, _dp_spec, _b_spec, _w_spec],
        out_specs=_row_spec, out_shape=_out_sds)(p, g1, dp, b1, W2)


def _tc3(p, g2, dp, b2):
    return pl.pallas_call(
        _tc3_body, grid=(_GRID,),
        in_specs=[_p_spec, _row_spec, _dp_spec, _b_spec],
        out_specs=_row_spec, out_shape=_out_sds)(p, g2, dp, b2)


def kernel(x, edge_index, W1, b1, W2, b2):
    ei = edge_index.astype(jnp.int32)
    src2d = ei[0].reshape(ROWS2D, CH)
    dst2d = ei[1].reshape(ROWS2D, CH)
    zrows = jnp.zeros((NACC, D), jnp.float32)
    zdeg = jnp.zeros((NDEG,), jnp.float32)
    b1r = b1.reshape(1, D)
    b2r = b2.reshape(1, D)

    degp = _sc_degree()(dst2d, zdeg)
    dp = degp.T

    g1 = _tc1(x, W1, dp)
    p1 = _sc_aggregate()(g1, src2d, dst2d, zrows)
    g2 = _tc2(p1, g1, dp, b1r, W2)
    p2 = _sc_aggregate()(g2, src2d, dst2d, zrows)
    return _tc3(p2, g2, dp, b2r)


# speedup vs baseline: 1.8195x; 1.0365x over previous
import functools

import jax
import jax.numpy as jnp
from jax import lax
from jax.experimental import pallas as pl
from jax.experimental.pallas import tpu as pltpu
from jax.experimental.pallas import tpu_sc as plsc

N = 10000
D = 128
E = 320000
NC = 2
NS = 16
NW = NC * NS
CH = 125
CPW = (E // NW) // CH
ROWS2D = E // CH
NDEG = 10240
DEG_SL = NDEG // NS
NACC = 10112
ACC_SL = NACC // NS

def _mesh():
    return plsc.VectorSubcoreMesh(
        core_axis_name="c", subcore_axis_name="s",
        num_cores=NC, num_subcores=NS)


def _sc_degree_body(ei_hbm, zero_hbm, out_hbm, idx_v, ones_v, acc):
    c = lax.axis_index("c")
    s = lax.axis_index("s")
    w = c * NS + s
    pltpu.sync_copy(zero_hbm.at[pl.ds(s * DEG_SL, DEG_SL)],
                    acc.at[pl.ds(s * DEG_SL, DEG_SL)])
    pltpu.sync_copy(ei_hbm.at[1, pl.ds(w * CPW, CPW)], idx_v)
    for k in range(8):
        ones_v[pl.ds(k * 16, 16)] = jnp.ones((16,), jnp.float32)
    plsc.subcore_barrier()

    def body(j, carry):
        pltpu.sync_copy(ones_v.at[pl.ds(0, CH)], acc.at[idx_v.at[j]], add=True)
        return carry

    lax.fori_loop(0, CPW, body, 0)
    plsc.subcore_barrier()
    pltpu.sync_copy(acc.at[pl.ds(s * DEG_SL, DEG_SL)],
                    out_hbm.at[c, pl.ds(s * DEG_SL, DEG_SL)])


NB = 2
NI = 4


def _sc_aggregate_body(tab_hbm, ei_hbm, zero_hbm, out_hbm,
                       sidx, didx, rows_v, acc,
                       is0, is1, is2, is3, gs0, gs1, ss0, ss1):
    isem = (is0, is1, is2, is3)
    gsem = (gs0, gs1)
    ssem = (ss0, ss1)
    src_hbm = ei_hbm.at[0]
    dst_hbm = ei_hbm.at[1]
    c = lax.axis_index("c")
    s = lax.axis_index("s")
    base = (c * NS + s) * CPW

    def start_i(sl, j):
        pltpu.async_copy(src_hbm.at[base + j], sidx.at[sl], isem[sl])
        pltpu.async_copy(dst_hbm.at[base + j], didx.at[sl], isem[sl])

    def wait_i(sl, j):
        pltpu.make_async_copy(src_hbm.at[base + j], sidx.at[sl],
                              isem[sl]).wait()
        pltpu.make_async_copy(dst_hbm.at[base + j], didx.at[sl],
                              isem[sl]).wait()

    def start_g(b, sl):
        pltpu.async_copy(tab_hbm.at[sidx.at[sl]], rows_v.at[b], gsem[b])

    def wait_g(b, sl):
        pltpu.make_async_copy(tab_hbm.at[sidx.at[sl]], rows_v.at[b],
                              gsem[b]).wait()

    def start_s(b, sl):
        pltpu.async_copy(rows_v.at[b], acc.at[didx.at[sl]], ssem[b],
                         add=True)

    def wait_s(b, sl):
        pltpu.make_async_copy(rows_v.at[b], acc.at[didx.at[sl]],
                              ssem[b]).wait()

    for k in range(NI):
        start_i(k, k)
    pltpu.sync_copy(zero_hbm.at[pl.ds(s * ACC_SL, ACC_SL)],
                    acc.at[pl.ds(s * ACC_SL, ACC_SL)])
    for b in range(NB):
        wait_i(b, b)
        start_g(b, b)
    plsc.subcore_barrier()

    def grp_body(grp, carry):
        for k in range(NI):
            j = grp * NI + k
            b = k % NB
            wait_g(b, k)
            start_s(b, k)
            wait_s(b, k)
            pl.when(j + NI < CPW)(lambda k=k, j=j: start_i(k, j + NI))

            def next_g(j=j, b=b, k=k):
                wait_i((k + NB) % NI, j + NB)
                start_g(b, (k + NB) % NI)

            pl.when(j + NB < CPW)(next_g)
        return carry

    lax.fori_loop(0, CPW // NI, grp_body, 0)
    plsc.subcore_barrier()
    pltpu.sync_copy(acc.at[pl.ds(s * ACC_SL, ACC_SL)],
                    out_hbm.at[c, pl.ds(s * ACC_SL, ACC_SL)])


@functools.cache
def _sc_degree():
    return pl.kernel(
        _sc_degree_body,
        out_type=jax.ShapeDtypeStruct((NC, NDEG), jnp.float32),
        mesh=_mesh(),
        scratch_types=[
            pltpu.VMEM((CPW, CH), jnp.int32),
            pltpu.VMEM((128,), jnp.float32),
            pltpu.VMEM_SHARED((NDEG,), jnp.float32),
        ],
    )


@functools.cache
def _sc_aggregate():
    return pl.kernel(
        _sc_aggregate_body,
        out_type=jax.ShapeDtypeStruct((NC, NACC, D), jnp.float32),
        mesh=_mesh(),
        scratch_types=[
            pltpu.VMEM((NI, CH), jnp.int32),
            pltpu.VMEM((NI, CH), jnp.int32),
            pltpu.VMEM((NB, CH, D), jnp.float32),
            pltpu.VMEM_SHARED((NACC, D), jnp.float32),
        ] + [pltpu.SemaphoreType.DMA] * (NI + 2 * NB),
    )


_BLK = 1000
_GRID = N // _BLK


def _dis(dp_ref):
    return lax.rsqrt(dp_ref[:, 0:1] + dp_ref[:, 1:2] + 1.0)


def _tc1_body(x_ref, w_ref, dp_ref, o_ref):
    o_ref[...] = jnp.dot(x_ref[...], w_ref[...],
                         preferred_element_type=jnp.float32) * _dis(dp_ref)


def _tc2_body(p_ref, g1_ref, dp_ref, b_ref, w_ref, o_ref):
    dis = _dis(dp_ref)
    h = jnp.maximum((p_ref[0] + p_ref[1] + g1_ref[...]) * dis + b_ref[...], 0.0)
    o_ref[...] = jnp.dot(h, w_ref[...],
                         preferred_element_type=jnp.float32) * dis


def _tc3_body(p_ref, g2_ref, dp_ref, b_ref, o_ref):
    o_ref[...] = (p_ref[0] + p_ref[1] + g2_ref[...]) * _dis(dp_ref) + b_ref[...]


_row_spec = pl.BlockSpec((_BLK, D), lambda i: (i, 0))
_dp_spec = pl.BlockSpec((_BLK, 2), lambda i: (i, 0))
_w_spec = pl.BlockSpec((D, D), lambda i: (0, 0))
_b_spec = pl.BlockSpec((1, D), lambda i: (0, 0))
_p_spec = pl.BlockSpec((NC, _BLK, D), lambda i: (0, i, 0))
_out_sds = jax.ShapeDtypeStruct((N, D), jnp.float32)


def _tc1(x, W1, dp):
    return pl.pallas_call(
        _tc1_body, grid=(_GRID,),
        in_specs=[_row_spec, _w_spec, _dp_spec],
        out_specs=_row_spec, out_shape=_out_sds)(x, W1, dp)


def _tc2(p, g1, dp, b1, W2):
    return pl.pallas_call(
        _tc2_body, grid=(_GRID,),
        in_specs=[_p_spec, _row_spec, _dp_spec, _b_spec, _w_spec],
        out_specs=_row_spec, out_shape=_out_sds)(p, g1, dp, b1, W2)


def _tc3(p, g2, dp, b2):
    return pl.pallas_call(
        _tc3_body, grid=(_GRID,),
        in_specs=[_p_spec, _row_spec, _dp_spec, _b_spec],
        out_specs=_row_spec, out_shape=_out_sds)(p, g2, dp, b2)


def kernel(x, edge_index, W1, b1, W2, b2):
    ei3 = edge_index.astype(jnp.int32).reshape(2, ROWS2D, CH)
    zrows = jnp.zeros((NACC, D), jnp.float32)
    zdeg = jnp.zeros((NDEG,), jnp.float32)
    b1r = b1.reshape(1, D)
    b2r = b2.reshape(1, D)

    degp = _sc_degree()(ei3, zdeg)
    dp = degp.T

    g1 = _tc1(x, W1, dp)
    p1 = _sc_aggregate()(g1, ei3, zrows)
    g2 = _tc2(p1, g1, dp, b1r, W2)
    p2 = _sc_aggregate()(g2, ei3, zrows)
    return _tc3(p2, g2, dp, b2r)
